# baseline (speedup 1.0000x reference)
import jax
import jax.numpy as jnp
from jax.experimental import pallas as pl
from jax.experimental.pallas import tpu as pltpu

_BI = 400
_FC_CHUNK = 1000


def _body(adj_ref, x_ref, w1_ref, b1_ref, w2_ref, b2_ref, w0_ref,
          o_ref, h1_ref, h2_ref):
    p = pl.program_id(0)
    i = pl.program_id(1)
    n = x_ref.shape[0]

    @pl.when((p == 0) & (i == 0))
    def _():
        w1 = w1_ref[...]
        b1 = b1_ref[...]
        for c in range(0, n, _FC_CHUNK):
            xc = x_ref[pl.ds(c, _FC_CHUNK), :]
            h1_ref[pl.ds(c, _FC_CHUNK), :] = (
                jnp.dot(xc, w1, preferred_element_type=jnp.float32) + b1
            )

    @pl.when(p == 0)
    def _():
        t = jnp.dot(adj_ref[...], h1_ref[...], preferred_element_type=jnp.float32)
        r = jnp.maximum(t, 0.0)
        v = jnp.dot(r, w2_ref[...], preferred_element_type=jnp.float32)
        h2_ref[pl.ds(i * _BI, _BI), :] = v + b2_ref[...]

    @pl.when(p == 1)
    def _():
        t = jnp.dot(adj_ref[...], h2_ref[...], preferred_element_type=jnp.float32)
        o_ref[...] = jnp.maximum(t, 0.0) * w0_ref[0, 0]


def kernel(seq1, adj, sparse, W1, b1, W2, b2, w0):
    del sparse
    n = seq1.shape[2]
    d_in = seq1.shape[3]
    d_h = W1.shape[1]
    d_out = W2.shape[1]
    x = seq1.reshape(n, d_in)
    a = adj.reshape(n, n)
    ni = n // _BI

    out = pl.pallas_call(
        _body,
        grid=(2, ni),
        in_specs=[
            pl.BlockSpec((_BI, n), lambda p, i: (i + p * (ni - 1 - 2 * i), 0)),
            pl.BlockSpec((n, d_in), lambda p, i: (0, 0)),
            pl.BlockSpec((d_in, d_h), lambda p, i: (0, 0)),
            pl.BlockSpec((1, d_h), lambda p, i: (0, 0)),
            pl.BlockSpec((d_h, d_out), lambda p, i: (0, 0)),
            pl.BlockSpec((1, d_out), lambda p, i: (0, 0)),
            pl.BlockSpec((1, 1), lambda p, i: (0, 0)),
        ],
        out_specs=pl.BlockSpec((_BI, d_out), lambda p, i: (p * (ni - 1 - i), 0)),
        out_shape=jax.ShapeDtypeStruct((n, d_out), jnp.float32),
        scratch_shapes=[
            pltpu.VMEM((n, d_h), jnp.float32),
            pltpu.VMEM((n, d_out), jnp.float32),
        ],
        compiler_params=pltpu.CompilerParams(
            dimension_semantics=("arbitrary", "parallel"),
            vmem_limit_bytes=64 * 1024 * 1024,
        ),
    )(a, x, W1, b1.reshape(1, d_h), W2, b2.reshape(1, d_out), w0.reshape(1, 1))

    return out.reshape(1, n, d_out)


# speedup vs baseline: 1.0032x; 1.0032x over previous
import jax
import jax.numpy as jnp
from jax.experimental import pallas as pl
from jax.experimental.pallas import tpu as pltpu

_BI = 400
_FC_CHUNK = 1000


def _body(adj_ref, x_ref, w1_ref, b1_ref, w2_ref, b2_ref, w0_ref,
          o_ref, h1_ref, h2_ref):
    p = pl.program_id(0)
    i = pl.program_id(1)
    n = x_ref.shape[0]

    @pl.when((p == 0) & (i == 0))
    def _():
        w1 = w1_ref[...]
        b1 = b1_ref[...]
        for c in range(0, n, _FC_CHUNK):
            xc = x_ref[pl.ds(c, _FC_CHUNK), :]
            h1_ref[pl.ds(c, _FC_CHUNK), :] = (
                jnp.dot(xc, w1, preferred_element_type=jnp.float32) + b1
            )

    @pl.when(p == 0)
    def _():
        t = jnp.dot(adj_ref[...], h1_ref[...], preferred_element_type=jnp.float32)
        r = jnp.maximum(t, 0.0)
        v = jnp.dot(r, w2_ref[...], preferred_element_type=jnp.float32)
        h2_ref[pl.ds(i * _BI, _BI), :] = v + b2_ref[...]

    @pl.when(p == 1)
    def _():
        t = jnp.dot(adj_ref[...], h2_ref[...], preferred_element_type=jnp.float32)
        o_ref[...] = jnp.maximum(t, 0.0) * w0_ref[0, 0]


def kernel(seq1, adj, sparse, W1, b1, W2, b2, w0):
    del sparse
    n = seq1.shape[2]
    d_in = seq1.shape[3]
    d_h = W1.shape[1]
    d_out = W2.shape[1]
    x = seq1.reshape(n, d_in)
    a = adj.reshape(n, n)
    ni = n // _BI

    out = pl.pallas_call(
        _body,
        grid=(2, ni),
        in_specs=[
            pl.BlockSpec((_BI, n), lambda p, i: (i, 0)),
            pl.BlockSpec((n, d_in), lambda p, i: (0, 0)),
            pl.BlockSpec((d_in, d_h), lambda p, i: (0, 0)),
            pl.BlockSpec((1, d_h), lambda p, i: (0, 0)),
            pl.BlockSpec((d_h, d_out), lambda p, i: (0, 0)),
            pl.BlockSpec((1, d_out), lambda p, i: (0, 0)),
            pl.BlockSpec((1, 1), lambda p, i: (0, 0)),
        ],
        out_specs=pl.BlockSpec((_BI, d_out), lambda p, i: (p * i, 0)),
        out_shape=jax.ShapeDtypeStruct((n, d_out), jnp.float32),
        scratch_shapes=[
            pltpu.VMEM((n, d_h), jnp.float32),
            pltpu.VMEM((n, d_out), jnp.float32),
        ],
        compiler_params=pltpu.CompilerParams(
            dimension_semantics=("arbitrary", "parallel"),
            vmem_limit_bytes=64 * 1024 * 1024,
        ),
    )(a, x, W1, b1.reshape(1, d_h), W2, b2.reshape(1, d_out), w0.reshape(1, 1))

    return out.reshape(1, n, d_out)
